# baseline (speedup 1.0000x reference)
import functools

import jax
import jax.numpy as jnp
from jax import lax
from jax.experimental import pallas as pl
from jax.experimental.pallas import tpu as pltpu
from jax.experimental.pallas import tpu_sc as plsc

B, N, D, K = 8, 2048, 32, 128
ROW = 256
BIG = 1e6
LOG2E = 1.4426950408889634

CHUNKS = 4
CHUNK = N // CHUNKS
FR = 40
FUSED = D + 3
GROUPS = CHUNK // 16

_dot = functools.partial(
    jax.lax.dot_general, preferred_element_type=jnp.float32
)


def _sc_body(featT_hbm, sid_hbm, out_hbm, featT_v, sid_v, tab_v):
    c = lax.axis_index("c")
    s = lax.axis_index("s")
    wid = c * 16 + s
    batch = wid // CHUNKS
    chunk = wid % CHUNKS
    n0 = chunk * CHUNK

    pltpu.sync_copy(featT_hbm.at[batch, :, pl.ds(n0, CHUNK)], featT_v)
    pltpu.sync_copy(sid_hbm.at[batch, pl.ds(n0, CHUNK)], sid_v)

    z16 = jnp.zeros((16,), jnp.float32)
    for i in range(FR * K // 16):
        tab_v[pl.ds(i * 16, 16)] = z16

    for g in range(GROUPS):
        g0 = g * 16
        base = sid_v[pl.ds(g0, 16)] * FR
        for f in range(FUSED):
            v = featT_v[f, pl.ds(g0, 16)]
            plsc.addupdate_scatter(tab_v, [base + f], v)

    pltpu.sync_copy(tab_v, out_hbm.at[batch, chunk])


@functools.lru_cache(maxsize=1)
def _get_sc_stats():
    return functools.partial(
        pl.kernel,
        out_type=jax.ShapeDtypeStruct((B, CHUNKS, K * FR), jnp.float32),
        mesh=plsc.VectorSubcoreMesh(core_axis_name="c", subcore_axis_name="s"),
        scratch_types=[
            pltpu.VMEM((FR, CHUNK), jnp.float32),
            pltpu.VMEM((CHUNK,), jnp.int32),
            pltpu.VMEM((FR * K,), jnp.float32),
        ],
        compiler_params=pltpu.CompilerParams(needs_layout_passes=False),
    )(_sc_body)


def _oc_kernel(beta_ref, emb_ref, sid_ref, cp_ref, cpc_ref, stats_ref,
               out_ref, acc_ref):
    b = pl.program_id(0)

    @pl.when(b == 0)
    def _init():
        acc_ref[0] = 0.0
        acc_ref[1] = 0.0

    sid = sid_ref[0]
    cp = cp_ref[0] == 1
    x = beta_ref[0]
    E = emb_ref[0]
    cpc = (cpc_ref[0] == 1).astype(jnp.float32)

    cpf = cp.astype(jnp.float32)
    pos = jnp.sum(cpf)
    neg = jnp.float32(N) - pos

    pos_w = neg / (pos + 1e-6)
    w = jnp.where(cp, pos_w, 1.0)
    bce = jnp.maximum(x, 0.0) - x * cpf + jnp.log1p(jnp.exp(-jnp.abs(x)))
    beta_loss = jnp.sum(w * bce) * (1.0 / jnp.float32(N))

    st = stats_ref[0]
    tab = ((st[0] + st[1]) + (st[2] + st[3]))
    S1 = tab[:, :D]
    Ssq = tab[:, D:D + 1]
    cnt = tab[:, D + 1:D + 2]
    has = (tab[:, D + 2:D + 3] > 0.0).astype(jnp.float32)

    kk = jax.lax.broadcasted_iota(jnp.int32, (K, N), 0)
    nn = jax.lax.broadcasted_iota(jnp.int32, (K, N), 1)
    sidm = jnp.where(cp, sid, -1)
    cpm = sidm == kk
    first = jnp.min(jnp.where(cpm, nn, N), axis=1, keepdims=True)
    Ff = (nn == first).astype(jnp.float32)

    sq_col = jnp.sum(E * E, axis=1, keepdims=True)
    A = jnp.concatenate([E, sq_col], axis=1)
    CA = _dot(Ff, A, (((1,), (0,)), ((), ())))
    C = CA[:, :D]
    csq = CA[:, D:D + 1]
    cross = jnp.sum(C * S1, axis=1, keepdims=True)
    safe_cnt = jnp.maximum(cnt, 1.0)
    terms = has * (Ssq + cnt * csq - 2.0 * cross) / safe_cnt
    attraction = jnp.sum(terms)

    ones_col = jnp.ones((N, 1), jnp.float32)
    na = -(sq_col + BIG * (1.0 - cpc))
    X = jnp.concatenate([(2.0 * LOG2E) * E, LOG2E * na, ones_col], axis=1)
    Y = jnp.concatenate([E, ones_col, LOG2E * na], axis=1)

    rep_sum = jnp.float32(0.0)
    for i in range(N // ROW):
        r0 = i * ROW
        Xi = X[r0:r0 + ROW, :]
        Yi = Y[r0:, :]
        m = _dot(Xi, Yi, (((1,), (1,)), ((), ())))
        e = jnp.exp2(m)
        rep_sum = rep_sum + jnp.sum(e[:, :ROW])
        if r0 + ROW < N:
            rep_sum = rep_sum + 2.0 * jnp.sum(e[:, ROW:])
    repulsion = jnp.where(pos > 1.0, rep_sum / (pos * pos), 0.0)

    loss_b = beta_loss + attraction + repulsion
    valid = (pos >= 1.0) & (neg >= 1.0)
    acc_ref[0] += jnp.where(valid, loss_b, 0.0)
    acc_ref[1] += valid.astype(jnp.float32)

    @pl.when(b == B - 1)
    def _fin():
        cnt_v = acc_ref[1]
        out_ref[0, 0] = jnp.where(cnt_v == 0.0, 0.0,
                                  acc_ref[0] / jnp.maximum(cnt_v, 1.0))


@jax.jit
def kernel(beta, embed, slice_id, is_cp):
    beta2 = jnp.reshape(beta, (B, 1, N))
    sid2 = jnp.reshape(slice_id, (B, 1, N))
    cp2 = jnp.reshape(is_cp, (B, 1, N))
    cpc = jnp.reshape(is_cp, (B, N, 1))
    sq = jnp.sum(embed * embed, axis=-1, keepdims=True)
    feat = jnp.concatenate(
        [embed, sq, jnp.ones((B, N, 1), jnp.float32),
         is_cp[..., None].astype(jnp.float32),
         jnp.zeros((B, N, FR - D - 3), jnp.float32)], axis=-1)
    featT = jnp.transpose(feat, (0, 2, 1))
    parts = _get_sc_stats()(featT, slice_id)
    stats = jnp.reshape(parts, (B, CHUNKS, K, FR))
    out = pl.pallas_call(
        _oc_kernel,
        grid=(B,),
        in_specs=[
            pl.BlockSpec((1, 1, N), lambda b: (b, 0, 0)),
            pl.BlockSpec((1, N, D), lambda b: (b, 0, 0)),
            pl.BlockSpec((1, 1, N), lambda b: (b, 0, 0)),
            pl.BlockSpec((1, 1, N), lambda b: (b, 0, 0)),
            pl.BlockSpec((1, N, 1), lambda b: (b, 0, 0)),
            pl.BlockSpec((1, CHUNKS, K, FR), lambda b: (b, 0, 0, 0)),
        ],
        out_specs=pl.BlockSpec(memory_space=pltpu.SMEM),
        out_shape=jax.ShapeDtypeStruct((1, 1), jnp.float32),
        scratch_shapes=[pltpu.SMEM((2,), jnp.float32)],
    )(beta2, embed, sid2, cp2, cpc, stats)
    return out[0, 0]


# speedup vs baseline: 1.0006x; 1.0006x over previous
import functools

import jax
import jax.numpy as jnp
from jax import lax
from jax.experimental import pallas as pl
from jax.experimental.pallas import tpu as pltpu
from jax.experimental.pallas import tpu_sc as plsc

B, N, D, K = 8, 2048, 32, 128
ROW = 256
BIG = 1e6
LOG2E = 1.4426950408889634

CHUNKS = 4
CHUNK = N // CHUNKS
FR = 36
GROUPS = CHUNK // 16

_dot = functools.partial(
    jax.lax.dot_general, preferred_element_type=jnp.float32
)


def _sc_body(embT_hbm, sid_hbm, cp_hbm, out_hbm, embT_v, sid_v, cp_v, tab_v):
    c = lax.axis_index("c")
    s = lax.axis_index("s")
    wid = c * 16 + s
    batch = wid // CHUNKS
    chunk = wid % CHUNKS
    n0 = chunk * CHUNK

    pltpu.sync_copy(embT_hbm.at[batch, :, pl.ds(n0, CHUNK)], embT_v)
    pltpu.sync_copy(sid_hbm.at[batch, pl.ds(n0, CHUNK)], sid_v)
    pltpu.sync_copy(cp_hbm.at[batch, pl.ds(n0, CHUNK)], cp_v)

    z16 = jnp.zeros((16,), jnp.float32)
    for i in range(FR * K // 16):
        tab_v[pl.ds(i * 16, 16)] = z16

    ones16 = jnp.ones((16,), jnp.float32)
    for g in range(GROUPS):
        g0 = g * 16
        base = sid_v[pl.ds(g0, 16)] * FR
        sq = z16
        for f in range(D):
            v = embT_v[f, pl.ds(g0, 16)]
            sq = sq + v * v
            plsc.addupdate_scatter(tab_v, [base + f], v)
        plsc.addupdate_scatter(tab_v, [base + D], sq)
        plsc.addupdate_scatter(tab_v, [base + (D + 1)], ones16)
        cpg = cp_v[pl.ds(g0, 16)].astype(jnp.float32)
        plsc.addupdate_scatter(tab_v, [base + (D + 2)], cpg)

    pltpu.sync_copy(tab_v, out_hbm.at[batch, chunk])


@functools.lru_cache(maxsize=1)
def _get_sc_stats():
    return functools.partial(
        pl.kernel,
        out_type=jax.ShapeDtypeStruct((B, CHUNKS, K * FR), jnp.float32),
        mesh=plsc.VectorSubcoreMesh(core_axis_name="c", subcore_axis_name="s"),
        scratch_types=[
            pltpu.VMEM((D, CHUNK), jnp.float32),
            pltpu.VMEM((CHUNK,), jnp.int32),
            pltpu.VMEM((CHUNK,), jnp.int32),
            pltpu.VMEM((FR * K,), jnp.float32),
        ],
        compiler_params=pltpu.CompilerParams(needs_layout_passes=False),
    )(_sc_body)


def _oc_kernel(beta_ref, emb_ref, sid_ref, cp_ref, cpc_ref, stats_ref,
               out_ref, acc_ref):
    b = pl.program_id(0)

    @pl.when(b == 0)
    def _init():
        acc_ref[0] = 0.0
        acc_ref[1] = 0.0

    sid = sid_ref[0]
    cp = cp_ref[0] == 1
    x = beta_ref[0]
    E = emb_ref[0]
    cpc = (cpc_ref[0] == 1).astype(jnp.float32)

    cpf = cp.astype(jnp.float32)
    pos = jnp.sum(cpf)
    neg = jnp.float32(N) - pos

    pos_w = neg / (pos + 1e-6)
    w = jnp.where(cp, pos_w, 1.0)
    bce = jnp.maximum(x, 0.0) - x * cpf + jnp.log1p(jnp.exp(-jnp.abs(x)))
    beta_loss = jnp.sum(w * bce) * (1.0 / jnp.float32(N))

    st = stats_ref[0]
    tab = ((st[0] + st[1]) + (st[2] + st[3]))
    S1 = tab[:, :D]
    Ssq = tab[:, D:D + 1]
    cnt = tab[:, D + 1:D + 2]
    has = (tab[:, D + 2:D + 3] > 0.0).astype(jnp.float32)

    kk = jax.lax.broadcasted_iota(jnp.int32, (K, N), 0)
    nn = jax.lax.broadcasted_iota(jnp.int32, (K, N), 1)
    sidm = jnp.where(cp, sid, -1)
    cpm = sidm == kk
    first = jnp.min(jnp.where(cpm, nn, N), axis=1, keepdims=True)
    Ff = (nn == first).astype(jnp.float32)

    sq_col = jnp.sum(E * E, axis=1, keepdims=True)
    A = jnp.concatenate([E, sq_col], axis=1)
    CA = _dot(Ff, A, (((1,), (0,)), ((), ())))
    C = CA[:, :D]
    csq = CA[:, D:D + 1]
    cross = jnp.sum(C * S1, axis=1, keepdims=True)
    safe_cnt = jnp.maximum(cnt, 1.0)
    terms = has * (Ssq + cnt * csq - 2.0 * cross) / safe_cnt
    attraction = jnp.sum(terms)

    ones_col = jnp.ones((N, 1), jnp.float32)
    na = -(sq_col + BIG * (1.0 - cpc))
    X = jnp.concatenate([(2.0 * LOG2E) * E, LOG2E * na, ones_col], axis=1)
    Y = jnp.concatenate([E, ones_col, LOG2E * na], axis=1)

    rep_sum = jnp.float32(0.0)
    for i in range(N // ROW):
        r0 = i * ROW
        Xi = X[r0:r0 + ROW, :]
        Yi = Y[r0:, :]
        m = _dot(Xi, Yi, (((1,), (1,)), ((), ())))
        e = jnp.exp2(m)
        rep_sum = rep_sum + jnp.sum(e[:, :ROW])
        if r0 + ROW < N:
            rep_sum = rep_sum + 2.0 * jnp.sum(e[:, ROW:])
    repulsion = jnp.where(pos > 1.0, rep_sum / (pos * pos), 0.0)

    loss_b = beta_loss + attraction + repulsion
    valid = (pos >= 1.0) & (neg >= 1.0)
    acc_ref[0] += jnp.where(valid, loss_b, 0.0)
    acc_ref[1] += valid.astype(jnp.float32)

    @pl.when(b == B - 1)
    def _fin():
        cnt_v = acc_ref[1]
        out_ref[0, 0] = jnp.where(cnt_v == 0.0, 0.0,
                                  acc_ref[0] / jnp.maximum(cnt_v, 1.0))


@jax.jit
def kernel(beta, embed, slice_id, is_cp):
    beta2 = jnp.reshape(beta, (B, 1, N))
    sid2 = jnp.reshape(slice_id, (B, 1, N))
    cp2 = jnp.reshape(is_cp, (B, 1, N))
    cpc = jnp.reshape(is_cp, (B, N, 1))
    embT = jnp.transpose(embed, (0, 2, 1))
    parts = _get_sc_stats()(embT, slice_id, is_cp)
    stats = jnp.reshape(parts, (B, CHUNKS, K, FR))
    out = pl.pallas_call(
        _oc_kernel,
        grid=(B,),
        in_specs=[
            pl.BlockSpec((1, 1, N), lambda b: (b, 0, 0)),
            pl.BlockSpec((1, N, D), lambda b: (b, 0, 0)),
            pl.BlockSpec((1, 1, N), lambda b: (b, 0, 0)),
            pl.BlockSpec((1, 1, N), lambda b: (b, 0, 0)),
            pl.BlockSpec((1, N, 1), lambda b: (b, 0, 0)),
            pl.BlockSpec((1, CHUNKS, K, FR), lambda b: (b, 0, 0, 0)),
        ],
        out_specs=pl.BlockSpec(memory_space=pltpu.SMEM),
        out_shape=jax.ShapeDtypeStruct((1, 1), jnp.float32),
        scratch_shapes=[pltpu.SMEM((2,), jnp.float32)],
    )(beta2, embed, sid2, cp2, cpc, stats)
    return out[0, 0]


# speedup vs baseline: 1.0860x; 1.0853x over previous
import functools

import jax
import jax.numpy as jnp
from jax import lax
from jax.experimental import pallas as pl
from jax.experimental.pallas import tpu as pltpu
from jax.experimental.pallas import tpu_sc as plsc

B, N, D, K = 8, 2048, 32, 128
ROW = 256
BIG = 1e6
LOG2E = 1.4426950408889634

CHUNKS = 4
CHUNK = N // CHUNKS
FR = 36
GROUPS = CHUNK // 16

_dot = functools.partial(
    jax.lax.dot_general, preferred_element_type=jnp.float32
)


def _sc_body(embT_hbm, sid_hbm, cp_hbm, out_hbm, embT_v, sid_v, cp_v, tab_v):
    c = lax.axis_index("c")
    s = lax.axis_index("s")
    wid = c * 16 + s
    batch = wid // CHUNKS
    chunk = wid % CHUNKS
    n0 = chunk * CHUNK

    pltpu.sync_copy(embT_hbm.at[batch, :, pl.ds(n0, CHUNK)], embT_v)
    pltpu.sync_copy(sid_hbm.at[batch, pl.ds(n0, CHUNK)], sid_v)
    pltpu.sync_copy(cp_hbm.at[batch, pl.ds(n0, CHUNK)], cp_v)

    z16 = jnp.zeros((16,), jnp.float32)
    for i in range(FR * K // 16):
        tab_v[pl.ds(i * 16, 16)] = z16

    ones16 = jnp.ones((16,), jnp.float32)
    for g in range(GROUPS):
        g0 = g * 16
        base = sid_v[pl.ds(g0, 16)] * FR
        sq = z16
        for f in range(D):
            v = embT_v[f, pl.ds(g0, 16)]
            sq = sq + v * v
            plsc.addupdate_scatter(tab_v, [base + f], v)
        plsc.addupdate_scatter(tab_v, [base + D], sq)
        plsc.addupdate_scatter(tab_v, [base + (D + 1)], ones16)
        cpg = cp_v[pl.ds(g0, 16)].astype(jnp.float32)
        plsc.addupdate_scatter(tab_v, [base + (D + 2)], cpg)

    pltpu.sync_copy(tab_v, out_hbm.at[batch, chunk])


@functools.lru_cache(maxsize=1)
def _get_sc_stats():
    return functools.partial(
        pl.kernel,
        out_type=jax.ShapeDtypeStruct((B, CHUNKS, K * FR), jnp.float32),
        mesh=plsc.VectorSubcoreMesh(core_axis_name="c", subcore_axis_name="s"),
        scratch_types=[
            pltpu.VMEM((D, CHUNK), jnp.float32),
            pltpu.VMEM((CHUNK,), jnp.int32),
            pltpu.VMEM((CHUNK,), jnp.int32),
            pltpu.VMEM((FR * K,), jnp.float32),
        ],
        compiler_params=pltpu.CompilerParams(needs_layout_passes=False),
    )(_sc_body)


def _tc_main(beta_ref, emb_ref, sid_ref, cp_ref, cpc_ref, ca_ref, part_ref):
    b = pl.program_id(0)

    sid = sid_ref[0]
    cp = cp_ref[0] == 1
    x = beta_ref[0]
    E = emb_ref[0]
    cpc = (cpc_ref[0] == 1).astype(jnp.float32)

    cpf = cp.astype(jnp.float32)
    pos = jnp.sum(cpf)
    neg = jnp.float32(N) - pos

    pos_w = neg / (pos + 1e-6)
    w = jnp.where(cp, pos_w, 1.0)
    bce = jnp.maximum(x, 0.0) - x * cpf + jnp.log1p(jnp.exp(-jnp.abs(x)))
    beta_loss = jnp.sum(w * bce) * (1.0 / jnp.float32(N))

    kk = jax.lax.broadcasted_iota(jnp.int32, (K, N), 0)
    nn = jax.lax.broadcasted_iota(jnp.int32, (K, N), 1)
    sidm = jnp.where(cp, sid, -1)
    cpm = sidm == kk
    first = jnp.min(jnp.where(cpm, nn, N), axis=1, keepdims=True)
    Ff = (nn == first).astype(jnp.float32)

    sq_col = jnp.sum(E * E, axis=1, keepdims=True)
    A = jnp.concatenate([E, sq_col], axis=1)
    ca_ref[0] = _dot(Ff, A, (((1,), (0,)), ((), ())))

    ones_col = jnp.ones((N, 1), jnp.float32)
    na = -(sq_col + BIG * (1.0 - cpc))
    X = jnp.concatenate([(2.0 * LOG2E) * E, LOG2E * na, ones_col], axis=1)
    Y = jnp.concatenate([E, ones_col, LOG2E * na], axis=1)

    rep_sum = jnp.float32(0.0)
    for i in range(N // ROW):
        r0 = i * ROW
        Xi = X[r0:r0 + ROW, :]
        Yi = Y[r0:, :]
        m = _dot(Xi, Yi, (((1,), (1,)), ((), ())))
        e = jnp.exp2(m)
        rep_sum = rep_sum + jnp.sum(e[:, :ROW])
        if r0 + ROW < N:
            rep_sum = rep_sum + 2.0 * jnp.sum(e[:, ROW:])
    repulsion = jnp.where(pos > 1.0, rep_sum / (pos * pos), 0.0)

    valid = (pos >= 1.0) & (neg >= 1.0)
    part_ref[b, 0] = beta_loss + repulsion
    part_ref[b, 1] = valid.astype(jnp.float32)


def _tc_combine(stats_ref, ca_ref, part_ref, out_ref):
    total = jnp.float32(0.0)
    count = jnp.float32(0.0)
    for b in range(B):
        st = stats_ref[b]
        tab = (st[0] + st[1]) + (st[2] + st[3])
        S1 = tab[:, :D]
        Ssq = tab[:, D:D + 1]
        cnt = tab[:, D + 1:D + 2]
        has = (tab[:, D + 2:D + 3] > 0.0).astype(jnp.float32)
        CA = ca_ref[b]
        C = CA[:, :D]
        csq = CA[:, D:D + 1]
        cross = jnp.sum(C * S1, axis=1, keepdims=True)
        safe_cnt = jnp.maximum(cnt, 1.0)
        terms = has * (Ssq + cnt * csq - 2.0 * cross) / safe_cnt
        attraction = jnp.sum(terms)
        valid = part_ref[b, 1]
        total = total + valid * (part_ref[b, 0] + attraction)
        count = count + valid
    out_ref[0, 0] = jnp.where(count == 0.0, 0.0,
                              total / jnp.maximum(count, 1.0))


@jax.jit
def kernel(beta, embed, slice_id, is_cp):
    beta2 = jnp.reshape(beta, (B, 1, N))
    sid2 = jnp.reshape(slice_id, (B, 1, N))
    cp2 = jnp.reshape(is_cp, (B, 1, N))
    cpc = jnp.reshape(is_cp, (B, N, 1))
    embT = jnp.transpose(embed, (0, 2, 1))
    parts = _get_sc_stats()(embT, slice_id, is_cp)
    stats = jnp.reshape(parts, (B, CHUNKS, K, FR))
    ca, part = pl.pallas_call(
        _tc_main,
        grid=(B,),
        in_specs=[
            pl.BlockSpec((1, 1, N), lambda b: (b, 0, 0)),
            pl.BlockSpec((1, N, D), lambda b: (b, 0, 0)),
            pl.BlockSpec((1, 1, N), lambda b: (b, 0, 0)),
            pl.BlockSpec((1, 1, N), lambda b: (b, 0, 0)),
            pl.BlockSpec((1, N, 1), lambda b: (b, 0, 0)),
        ],
        out_specs=[
            pl.BlockSpec((1, K, D + 1), lambda b: (b, 0, 0)),
            pl.BlockSpec(memory_space=pltpu.SMEM),
        ],
        out_shape=[
            jax.ShapeDtypeStruct((B, K, D + 1), jnp.float32),
            jax.ShapeDtypeStruct((B, 2), jnp.float32),
        ],
    )(beta2, embed, sid2, cp2, cpc)
    out = pl.pallas_call(
        _tc_combine,
        in_specs=[
            pl.BlockSpec(),
            pl.BlockSpec(),
            pl.BlockSpec(memory_space=pltpu.SMEM),
        ],
        out_specs=pl.BlockSpec(memory_space=pltpu.SMEM),
        out_shape=jax.ShapeDtypeStruct((1, 1), jnp.float32),
    )(stats, ca, part)
    return out[0, 0]
